# baseline (speedup 1.0000x reference)
import functools

import jax
import jax.numpy as jnp
from jax import lax
from jax.experimental import pallas as pl
from jax.experimental.pallas import tpu as pltpu
from jax.experimental.pallas import tpu_sc as plsc

N = 8192
K = 16
L = 16
NSUB = 32
QPW = N // NSUB
NCHUNK = N // L
CBUF = N


def _sqrt_nr(x):
    xs = jnp.maximum(x, jnp.float32(1e-35))
    i = lax.bitcast_convert_type(xs, jnp.int32)
    i = jnp.int32(0x5F3759DF) - lax.shift_right_arithmetic(i, 1)
    y = lax.bitcast_convert_type(i, jnp.float32)
    for _ in range(3):
        y = y * (jnp.float32(1.5) - jnp.float32(0.5) * xs * y * y)
    return x * y


def _merge16(bd, bi, d2, ci):
    sd, si = plsc.sort_key_val(d2, ci)
    rb_d = lax.rev(bd, (0,))
    rb_i = lax.rev(bi, (0,))
    sel = sd < rb_d
    md = jnp.where(sel, sd, rb_d)
    mi = jnp.where(sel, si, rb_i)
    nd, ni = plsc.sort_key_val(md, mi)
    return nd, ni


@functools.partial(
    pl.kernel,
    out_type=[jax.ShapeDtypeStruct((N * K,), jnp.float32)] * 4,
    mesh=plsc.VectorSubcoreMesh(core_axis_name="c", subcore_axis_name="s"),
    scratch_types=(
        [pltpu.VMEM((N,), jnp.float32)] * 3
        + [pltpu.VMEM((QPW,), jnp.float32)] * 3
        + [pltpu.VMEM((QPW * K,), jnp.float32)] * 4
        + [pltpu.VMEM((QPW * K,), jnp.int32)]
        + [pltpu.VMEM((CBUF + L,), jnp.float32),
           pltpu.VMEM((CBUF + L,), jnp.int32),
           pltpu.SemaphoreType.DMA]
    ),
    compiler_params=pltpu.CompilerParams(needs_layout_passes=False),
)
def _knn_sc(qx_h, qy_h, qz_h, kx_h, ky_h, kz_h,
            d_out, x_out, y_out, z_out,
            kx_v, ky_v, kz_v, qx_v, qy_v, qz_v,
            d_buf, x_buf, y_buf, z_buf, i_buf,
            cd_buf, ci_buf, sem):
    wid = lax.axis_index("s") * 2 + lax.axis_index("c")
    base = wid * QPW

    pltpu.sync_copy(kx_h, kx_v)
    pltpu.sync_copy(ky_h, ky_v)
    pltpu.sync_copy(kz_h, kz_v)
    pltpu.sync_copy(qx_h.at[pl.ds(base, QPW)], qx_v)
    pltpu.sync_copy(qy_h.at[pl.ds(base, QPW)], qy_v)
    pltpu.sync_copy(qz_h.at[pl.ds(base, QPW)], qz_v)

    iota = lax.iota(jnp.int32, L)
    inf_v = jnp.full((L,), jnp.inf, jnp.float32)

    def per_query(q, carry):
        qi = jnp.broadcast_to(q, (L,))
        qxs = plsc.load_gather(qx_v, [qi])
        qys = plsc.load_gather(qy_v, [qi])
        qzs = plsc.load_gather(qz_v, [qi])

        def dist2(c):
            kxv = kx_v[pl.ds(c * L, L)]
            kyv = ky_v[pl.ds(c * L, L)]
            kzv = kz_v[pl.ds(c * L, L)]
            dx = qxs - kxv
            dy = qys - kyv
            dz = qzs - kzv
            return dx * dx + dy * dy + dz * dz

        def pa(c, m):
            return jnp.minimum(m, dist2(c))

        m = lax.fori_loop(0, NCHUNK, pa, inf_v, unroll=8)
        ms, _ = plsc.sort_key_val(m, iota)
        tau = jnp.broadcast_to(ms[L - 1], (L,))

        def pb(c, off):
            d2 = dist2(c)
            hit = d2 <= tau
            hi = jnp.where(hit, jnp.int32(1), jnp.int32(0))
            pos = off + plsc.cumsum(hi) - 1
            plsc.store_scatter(cd_buf, [pos], d2, mask=hit)
            plsc.store_scatter(ci_buf, [pos], c * L + iota, mask=hit)
            cnt = plsc.all_reduce_population_count(hit)
            return off + cnt

        zero_v = jnp.zeros((L,), jnp.int32)
        off = lax.fori_loop(0, NCHUNK, pb, zero_v, unroll=8)
        cnt_end = off[0]
        cd_buf[pl.ds(cnt_end, L)] = inf_v

        nch = lax.div(cnt_end + (L - 1), jnp.int32(L))

        def pc(c, st):
            bd, bi, tau_c = st
            d2 = cd_buf[pl.ds(c * L, L)]
            ci = ci_buf[pl.ds(c * L, L)]
            hit = d2 < tau_c

            def do_merge(args):
                bd, bi, _ = args
                nd, ni = _merge16(bd, bi, d2, ci)
                return nd, ni, jnp.broadcast_to(nd[L - 1], (L,))

            return lax.cond(jnp.any(hit), do_merge, lambda a: a,
                            (bd, bi, tau_c))

        bd, bi, _ = lax.fori_loop(0, nch, pc, (inf_v, iota, inf_v))

        d_buf[pl.ds(q * K, K)] = _sqrt_nr(bd)
        i_buf[pl.ds(q * K, K)] = bi
        return carry

    lax.fori_loop(0, QPW, per_query, 0)

    pltpu.async_copy(kx_h.at[i_buf], x_buf, sem).wait()
    pltpu.async_copy(ky_h.at[i_buf], y_buf, sem).wait()
    pltpu.async_copy(kz_h.at[i_buf], z_buf, sem).wait()

    pltpu.sync_copy(d_buf, d_out.at[pl.ds(base * K, QPW * K)])
    pltpu.sync_copy(x_buf, x_out.at[pl.ds(base * K, QPW * K)])
    pltpu.sync_copy(y_buf, y_out.at[pl.ds(base * K, QPW * K)])
    pltpu.sync_copy(z_buf, z_out.at[pl.ds(base * K, QPW * K)])


def kernel(pcl_query, pcl_key):
    qt = pcl_query.T
    kt = pcl_key.T
    d, x, y, z = _knn_sc(qt[0], qt[1], qt[2], kt[0], kt[1], kt[2])
    dists = d.reshape(N, K)
    pcl = jnp.stack([x.reshape(N, K), y.reshape(N, K), z.reshape(N, K)],
                    axis=-1)
    return (pcl, dists)


# speedup vs baseline: 3.4633x; 3.4633x over previous
import functools

import jax
import jax.numpy as jnp
from jax import lax
from jax.experimental import pallas as pl
from jax.experimental.pallas import tpu as pltpu
from jax.experimental.pallas import tpu_sc as plsc

N = 8192
K = 16
L = 16
NSUB = 32
QPW = N // NSUB
NCHUNK = N // L
CBUF = N


def _sqrt_nr(x):
    xs = jnp.maximum(x, jnp.float32(1e-35))
    i = lax.bitcast_convert_type(xs, jnp.int32)
    i = jnp.int32(0x5F3759DF) - lax.shift_right_arithmetic(i, 1)
    y = lax.bitcast_convert_type(i, jnp.float32)
    for _ in range(3):
        y = y * (jnp.float32(1.5) - jnp.float32(0.5) * xs * y * y)
    return x * y


def _merge16(bd, bi, d2, ci):
    sd, si = plsc.sort_key_val(d2, ci)
    rb_d = lax.rev(bd, (0,))
    rb_i = lax.rev(bi, (0,))
    sel = sd < rb_d
    md = jnp.where(sel, sd, rb_d)
    mi = jnp.where(sel, si, rb_i)
    nd, ni = plsc.sort_key_val(md, mi)
    return nd, ni


@functools.partial(
    pl.kernel,
    out_type=[jax.ShapeDtypeStruct((N * K,), jnp.float32)] * 4,
    mesh=plsc.VectorSubcoreMesh(core_axis_name="c", subcore_axis_name="s"),
    scratch_types=(
        [pltpu.VMEM((N,), jnp.float32)] * 3
        + [pltpu.VMEM((QPW,), jnp.float32)] * 3
        + [pltpu.VMEM((QPW * K,), jnp.float32)] * 4
        + [pltpu.VMEM((QPW * K,), jnp.int32)]
        + [pltpu.VMEM((CBUF + L,), jnp.float32),
           pltpu.VMEM((CBUF + L,), jnp.int32),
           pltpu.SemaphoreType.DMA]
    ),
    compiler_params=pltpu.CompilerParams(needs_layout_passes=False),
)
def _knn_sc(qx_h, qy_h, qz_h, kx_h, ky_h, kz_h,
            d_out, x_out, y_out, z_out,
            kx_v, ky_v, kz_v, qx_v, qy_v, qz_v,
            d_buf, x_buf, y_buf, z_buf, i_buf,
            cd_buf, ci_buf, sem):
    wid = lax.axis_index("s") * 2 + lax.axis_index("c")
    base = wid * QPW

    pltpu.sync_copy(kx_h, kx_v)
    pltpu.sync_copy(ky_h, ky_v)
    pltpu.sync_copy(kz_h, kz_v)
    pltpu.sync_copy(qx_h.at[pl.ds(base, QPW)], qx_v)
    pltpu.sync_copy(qy_h.at[pl.ds(base, QPW)], qy_v)
    pltpu.sync_copy(qz_h.at[pl.ds(base, QPW)], qz_v)

    iota = lax.iota(jnp.int32, L)
    inf_v = jnp.full((L,), jnp.inf, jnp.float32)

    def per_query(q, carry):
        qi = jnp.broadcast_to(q, (L,))
        qxs = plsc.load_gather(qx_v, [qi])
        qys = plsc.load_gather(qy_v, [qi])
        qzs = plsc.load_gather(qz_v, [qi])

        def dist2(c):
            kxv = kx_v[pl.ds(c * L, L)]
            kyv = ky_v[pl.ds(c * L, L)]
            kzv = kz_v[pl.ds(c * L, L)]
            dx = qxs - kxv
            dy = qys - kyv
            dz = qzs - kzv
            return dx * dx + dy * dy + dz * dz

        @plsc.parallel_loop(0, NCHUNK, unroll=8, carry=inf_v)
        def pa_min(c, m):
            return jnp.minimum(m, dist2(c))

        m = pa_min
        ms, _ = plsc.sort_key_val(m, iota)
        tau = jnp.broadcast_to(ms[L - 1], (L,))

        zero_v = jnp.zeros((L,), jnp.int32)

        @plsc.parallel_loop(0, NCHUNK, unroll=8, carry=zero_v)
        def pb_scan(c, off):
            d2 = dist2(c)
            hit = d2 <= tau
            hi = jnp.where(hit, jnp.int32(1), jnp.int32(0))
            pos = off + plsc.cumsum(hi) - 1
            plsc.store_scatter(cd_buf, [pos], d2, mask=hit)
            plsc.store_scatter(ci_buf, [pos], c * L + iota, mask=hit)
            cnt = plsc.all_reduce_population_count(hit)
            return off + cnt

        off = pb_scan
        cnt_end = off[0]
        cd_buf[pl.ds(cnt_end, L)] = inf_v

        nch = lax.div(cnt_end + (L - 1), jnp.int32(L))

        def pc(c, st):
            bd, bi, tau_c = st
            d2 = cd_buf[pl.ds(c * L, L)]
            ci = ci_buf[pl.ds(c * L, L)]
            hit = d2 < tau_c

            def do_merge(args):
                bd, bi, _ = args
                nd, ni = _merge16(bd, bi, d2, ci)
                return nd, ni, jnp.broadcast_to(nd[L - 1], (L,))

            return lax.cond(jnp.any(hit), do_merge, lambda a: a,
                            (bd, bi, tau_c))

        bd, bi, _ = lax.fori_loop(0, nch, pc, (inf_v, iota, inf_v))

        d_buf[pl.ds(q * K, K)] = _sqrt_nr(bd)
        i_buf[pl.ds(q * K, K)] = bi
        return carry

    lax.fori_loop(0, QPW, per_query, 0)

    pltpu.async_copy(kx_h.at[i_buf], x_buf, sem).wait()
    pltpu.async_copy(ky_h.at[i_buf], y_buf, sem).wait()
    pltpu.async_copy(kz_h.at[i_buf], z_buf, sem).wait()

    pltpu.sync_copy(d_buf, d_out.at[pl.ds(base * K, QPW * K)])
    pltpu.sync_copy(x_buf, x_out.at[pl.ds(base * K, QPW * K)])
    pltpu.sync_copy(y_buf, y_out.at[pl.ds(base * K, QPW * K)])
    pltpu.sync_copy(z_buf, z_out.at[pl.ds(base * K, QPW * K)])


def kernel(pcl_query, pcl_key):
    qt = pcl_query.T
    kt = pcl_key.T
    d, x, y, z = _knn_sc(qt[0], qt[1], qt[2], kt[0], kt[1], kt[2])
    dists = d.reshape(N, K)
    pcl = jnp.stack([x.reshape(N, K), y.reshape(N, K), z.reshape(N, K)],
                    axis=-1)
    return (pcl, dists)


# speedup vs baseline: 4.7789x; 1.3799x over previous
import functools

import jax
import jax.numpy as jnp
from jax import lax
from jax.experimental import pallas as pl
from jax.experimental.pallas import tpu as pltpu
from jax.experimental.pallas import tpu_sc as plsc

N = 8192
K = 16
L = 16
NSUB = 32
QPW = N // NSUB
NCHUNK = N // L
CBUF = N


def _sqrt_nr(x):
    xs = jnp.maximum(x, jnp.float32(1e-35))
    i = lax.bitcast_convert_type(xs, jnp.int32)
    i = jnp.int32(0x5F3759DF) - lax.shift_right_arithmetic(i, 1)
    y = lax.bitcast_convert_type(i, jnp.float32)
    for _ in range(3):
        y = y * (jnp.float32(1.5) - jnp.float32(0.5) * xs * y * y)
    return x * y


def _merge16(bd, bi, d2, ci):
    sd, si = plsc.sort_key_val(d2, ci)
    rb_d = lax.rev(bd, (0,))
    rb_i = lax.rev(bi, (0,))
    sel = sd < rb_d
    md = jnp.where(sel, sd, rb_d)
    mi = jnp.where(sel, si, rb_i)
    nd, ni = plsc.sort_key_val(md, mi)
    return nd, ni


@functools.partial(
    pl.kernel,
    out_type=[jax.ShapeDtypeStruct((N * K,), jnp.float32)] * 4,
    mesh=plsc.VectorSubcoreMesh(core_axis_name="c", subcore_axis_name="s"),
    scratch_types=(
        [pltpu.VMEM((N,), jnp.float32)] * 3
        + [pltpu.VMEM((QPW,), jnp.float32)] * 3
        + [pltpu.VMEM((QPW * K,), jnp.float32)] * 4
        + [pltpu.VMEM((QPW * K,), jnp.int32)]
        + [pltpu.VMEM((CBUF + L,), jnp.float32),
           pltpu.VMEM((CBUF + L,), jnp.int32),
           pltpu.VMEM((N,), jnp.float32),
           pltpu.SemaphoreType.DMA]
    ),
    compiler_params=pltpu.CompilerParams(needs_layout_passes=False),
)
def _knn_sc(qx_h, qy_h, qz_h, kx_h, ky_h, kz_h,
            d_out, x_out, y_out, z_out,
            kx_v, ky_v, kz_v, qx_v, qy_v, qz_v,
            d_buf, x_buf, y_buf, z_buf, i_buf,
            cd_buf, ci_buf, d2_buf, sem):
    wid = lax.axis_index("s") * 2 + lax.axis_index("c")
    base = wid * QPW

    pltpu.sync_copy(kx_h, kx_v)
    pltpu.sync_copy(ky_h, ky_v)
    pltpu.sync_copy(kz_h, kz_v)
    pltpu.sync_copy(qx_h.at[pl.ds(base, QPW)], qx_v)
    pltpu.sync_copy(qy_h.at[pl.ds(base, QPW)], qy_v)
    pltpu.sync_copy(qz_h.at[pl.ds(base, QPW)], qz_v)

    iota = lax.iota(jnp.int32, L)
    inf_v = jnp.full((L,), jnp.inf, jnp.float32)

    def per_query(q, carry):
        qi = jnp.broadcast_to(q, (L,))
        qxs = plsc.load_gather(qx_v, [qi])
        qys = plsc.load_gather(qy_v, [qi])
        qzs = plsc.load_gather(qz_v, [qi])

        def dist2(c):
            kxv = kx_v[pl.ds(c * L, L)]
            kyv = ky_v[pl.ds(c * L, L)]
            kzv = kz_v[pl.ds(c * L, L)]
            dx = qxs - kxv
            dy = qys - kyv
            dz = qzs - kzv
            return dx * dx + dy * dy + dz * dz

        @plsc.parallel_loop(0, NCHUNK, unroll=8, carry=inf_v)
        def pa_min(c, m):
            d2 = dist2(c)
            d2_buf[pl.ds(c * L, L)] = d2
            return jnp.minimum(m, d2)

        m = pa_min
        ms, _ = plsc.sort_key_val(m, iota)
        tau = jnp.broadcast_to(ms[L - 1], (L,))

        zero_v = jnp.zeros((L,), jnp.int32)

        @plsc.parallel_loop(0, NCHUNK, unroll=8, carry=zero_v)
        def pb_scan(c, off):
            d2 = d2_buf[pl.ds(c * L, L)]
            hit = d2 <= tau
            hi = jnp.where(hit, jnp.int32(1), jnp.int32(0))
            pos = off + plsc.cumsum(hi) - 1
            plsc.store_scatter(cd_buf, [pos], d2, mask=hit)
            plsc.store_scatter(ci_buf, [pos], c * L + iota, mask=hit)
            cnt = plsc.all_reduce_population_count(hit)
            return off + cnt

        off = pb_scan
        cnt_end = off[0]
        cd_buf[pl.ds(cnt_end, L)] = inf_v

        nch = lax.div(cnt_end + (L - 1), jnp.int32(L))

        def pc(c, st):
            bd, bi, tau_c = st
            d2 = cd_buf[pl.ds(c * L, L)]
            ci = ci_buf[pl.ds(c * L, L)]
            hit = d2 < tau_c

            def do_merge(args):
                bd, bi, _ = args
                nd, ni = _merge16(bd, bi, d2, ci)
                return nd, ni, jnp.broadcast_to(nd[L - 1], (L,))

            return lax.cond(jnp.any(hit), do_merge, lambda a: a,
                            (bd, bi, tau_c))

        bd, bi, _ = lax.fori_loop(0, nch, pc, (inf_v, iota, inf_v))

        d_buf[pl.ds(q * K, K)] = _sqrt_nr(bd)
        i_buf[pl.ds(q * K, K)] = bi
        return carry

    lax.fori_loop(0, QPW, per_query, 0)

    pltpu.async_copy(kx_h.at[i_buf], x_buf, sem).wait()
    pltpu.async_copy(ky_h.at[i_buf], y_buf, sem).wait()
    pltpu.async_copy(kz_h.at[i_buf], z_buf, sem).wait()

    pltpu.sync_copy(d_buf, d_out.at[pl.ds(base * K, QPW * K)])
    pltpu.sync_copy(x_buf, x_out.at[pl.ds(base * K, QPW * K)])
    pltpu.sync_copy(y_buf, y_out.at[pl.ds(base * K, QPW * K)])
    pltpu.sync_copy(z_buf, z_out.at[pl.ds(base * K, QPW * K)])


def kernel(pcl_query, pcl_key):
    qt = pcl_query.T
    kt = pcl_key.T
    d, x, y, z = _knn_sc(qt[0], qt[1], qt[2], kt[0], kt[1], kt[2])
    dists = d.reshape(N, K)
    pcl = jnp.stack([x.reshape(N, K), y.reshape(N, K), z.reshape(N, K)],
                    axis=-1)
    return (pcl, dists)
